# baseline (speedup 1.0000x reference)
import jax
import jax.numpy as jnp
from jax.experimental import pallas as pl
from jax.experimental.pallas import tpu as pltpu
from jax.experimental.pallas import tpu_sc as plsc

D = 64
LANES = 16
CHUNK = 128
SCALE = 8.0


def _sc_embed(x_flat, table):
    n = x_flat.shape[1]
    mesh = plsc.VectorSubcoreMesh(core_axis_name="c", subcore_axis_name="s")

    @pl.kernel(
        out_type=jax.ShapeDtypeStruct((n, D), jnp.float32),
        mesh=mesh,
        compiler_params=pltpu.CompilerParams(use_tc_tiling_on_sc=False),
    )
    def k(table_hbm, idx_hbm, out_hbm):
        def body(idx_vmem, out_vmem):
            pltpu.sync_copy(table_hbm.at[idx_vmem.at[0]], out_vmem)

            @pl.loop(0, CHUNK)
            def _(r):
                @pl.loop(0, D, step=LANES)
                def _(j):
                    slc = out_vmem.at[r, pl.ds(j, LANES)]
                    slc[...] = slc[...] * SCALE

        pltpu.emit_pipeline(
            body,
            grid=(n // CHUNK,),
            in_specs=[pl.BlockSpec((1, CHUNK), lambda i: (0, i))],
            out_specs=[pl.BlockSpec((CHUNK, D), lambda i: (i, 0))],
            core_axis_name=("c", "s"),
            dimension_semantics=(pltpu.PARALLEL,),
        )(idx_hbm, out_hbm)

    return k(table, x_flat)


def kernel(x, table):
    b, s = x.shape
    x_flat = x.reshape(1, b * s).astype(jnp.int32)
    out = _sc_embed(x_flat, table)
    return out.reshape(b, s, D)


# speedup vs baseline: 1.0371x; 1.0371x over previous
import jax
import jax.numpy as jnp
from jax.experimental import pallas as pl
from jax.experimental.pallas import tpu as pltpu
from jax.experimental.pallas import tpu_sc as plsc

D = 64
LANES = 16
CHUNK = 512
SCALE = 8.0


def _sc_embed(x_flat, table):
    n = x_flat.shape[1]
    mesh = plsc.VectorSubcoreMesh(core_axis_name="c", subcore_axis_name="s")

    @pl.kernel(
        out_type=jax.ShapeDtypeStruct((n, D), jnp.float32),
        mesh=mesh,
        compiler_params=pltpu.CompilerParams(use_tc_tiling_on_sc=False),
    )
    def k(table_hbm, idx_hbm, out_hbm):
        def body(idx_vmem, out_vmem):
            pltpu.sync_copy(table_hbm.at[idx_vmem.at[0]], out_vmem)

            @pl.loop(0, CHUNK)
            def _(r):
                @pl.loop(0, D, step=LANES)
                def _(j):
                    slc = out_vmem.at[r, pl.ds(j, LANES)]
                    slc[...] = slc[...] * SCALE

        pltpu.emit_pipeline(
            body,
            grid=(n // CHUNK,),
            in_specs=[pl.BlockSpec((1, CHUNK), lambda i: (0, i))],
            out_specs=[pl.BlockSpec((CHUNK, D), lambda i: (i, 0))],
            core_axis_name=("c", "s"),
            dimension_semantics=(pltpu.PARALLEL,),
        )(idx_hbm, out_hbm)

    return k(table, x_flat)


def kernel(x, table):
    b, s = x.shape
    x_flat = x.reshape(1, b * s).astype(jnp.int32)
    out = _sc_embed(x_flat, table)
    return out.reshape(b, s, D)


# speedup vs baseline: 1.4802x; 1.4272x over previous
import functools

import jax
import jax.numpy as jnp
from jax.experimental import pallas as pl
from jax.experimental.pallas import tpu as pltpu
from jax.experimental.pallas import tpu_sc as plsc

D = 64
LANES = 16
CHUNK = 256
NC = 2
NS = 16
NW = NC * NS
SCALE = 8.0


def _sc_embed(x_flat, table):
    n = x_flat.shape[0]
    per_w = n // NW
    nchunks = per_w // CHUNK
    assert nchunks >= 4 and nchunks % 2 == 0
    mesh = plsc.VectorSubcoreMesh(core_axis_name="c", subcore_axis_name="s")

    @functools.partial(
        pl.kernel,
        out_type=jax.ShapeDtypeStruct((n, D), jnp.float32),
        mesh=mesh,
        compiler_params=pltpu.CompilerParams(use_tc_tiling_on_sc=False),
        scratch_types=[
            pltpu.VMEM((per_w,), jnp.int32),
            pltpu.VMEM((CHUNK, D), jnp.float32),
            pltpu.VMEM((CHUNK, D), jnp.float32),
            pltpu.VMEM((CHUNK, D), jnp.float32),
            pltpu.VMEM((CHUNK, D), jnp.float32),
            pltpu.SemaphoreType.DMA,
            pltpu.SemaphoreType.DMA,
            pltpu.SemaphoreType.DMA,
            pltpu.SemaphoreType.DMA,
        ],
    )
    def k(table_hbm, idx_hbm, out_hbm, idx_v, gb0, gb1, sb0, sb1,
          gsem0, gsem1, ssem0, ssem1):
        wid = jax.lax.axis_index("s") * NC + jax.lax.axis_index("c")
        base = wid * per_w
        pltpu.sync_copy(idx_hbm.at[pl.ds(base, per_w)], idx_v)

        gbufs = (gb0, gb1)
        sbufs = (sb0, sb1)
        gsems = (gsem0, gsem1)
        ssems = (ssem0, ssem1)

        def gather(g, b):
            return pltpu.make_async_copy(
                table_hbm.at[idx_v.at[pl.ds(g * CHUNK, CHUNK)]],
                gbufs[b], gsems[b])

        def store(g, b):
            return pltpu.make_async_copy(
                sbufs[b], out_hbm.at[pl.ds(base + g * CHUNK, CHUNK)],
                ssems[b])

        def scale(b):
            gb, sb = gbufs[b], sbufs[b]

            @pl.loop(0, CHUNK, step=8)
            def _(r0):
                for r in range(8):
                    for j in range(0, D, LANES):
                        sb.at[r0 + r, pl.ds(j, LANES)][...] = (
                            gb.at[r0 + r, pl.ds(j, LANES)][...] * SCALE)

        gather(0, 0).start()
        gather(1, 1).start()
        for b in range(2):
            gather(b, b).wait()
            scale(b)
            store(b, b).start()
            gather(b + 2, b).start()

        @pl.loop(2, nchunks - 2, step=2)
        def _(g):
            for b in range(2):
                gather(g + b, b).wait()
                store(g + b - 2, b).wait()
                scale(b)
                store(g + b, b).start()
                gather(g + b + 2, b).start()

        for b in range(2):
            g = nchunks - 2 + b
            gather(g, b).wait()
            store(g - 2, b).wait()
            scale(b)
            store(g, b).start()
        for b in range(2):
            store(nchunks - 2 + b, b).wait()

    return k(table, x_flat)


def kernel(x, table):
    b, s = x.shape
    x_flat = x.reshape(b * s).astype(jnp.int32)
    out = _sc_embed(x_flat, table)
    return out.reshape(b, s, D)
